# Initial kernel scaffold; baseline (speedup 1.0000x reference)
import jax
import jax.numpy as jnp
from jax.experimental import pallas as pl


def kernel(x, pos_table):
    raise NotImplementedError("write your pallas kernel here")


# speedup vs baseline: 1.3688x; 1.3688x over previous
import jax
import jax.numpy as jnp
from jax.experimental import pallas as pl

BATCH = 4
SEQ_LEN = 8192
D_MODEL = 768
BS = 1024


def _add_body(x_ref, pos_ref, out_ref):
    out_ref[...] = x_ref[...] + pos_ref[...][None]


def kernel(x, pos_table):
    grid = (BATCH, SEQ_LEN // BS)
    return pl.pallas_call(
        _add_body,
        grid=grid,
        in_specs=[
            pl.BlockSpec((1, BS, D_MODEL), lambda b, s: (b, s, 0)),
            pl.BlockSpec((BS, D_MODEL), lambda b, s: (s, 0)),
        ],
        out_specs=pl.BlockSpec((1, BS, D_MODEL), lambda b, s: (b, s, 0)),
        out_shape=jax.ShapeDtypeStruct((BATCH, SEQ_LEN, D_MODEL), jnp.float32),
    )(x, pos_table)


# speedup vs baseline: 1.6796x; 1.2271x over previous
import jax
import jax.numpy as jnp
from jax.experimental import pallas as pl

BATCH = 4
SEQ_LEN = 8192
D_MODEL = 768
BS = 1024


def _add_body(x_ref, pos_ref, out_ref):
    out_ref[...] = x_ref[...] + pos_ref[...][None]


def kernel(x, pos_table):
    grid = (SEQ_LEN // BS, BATCH)
    return pl.pallas_call(
        _add_body,
        grid=grid,
        in_specs=[
            pl.BlockSpec((1, BS, D_MODEL), lambda s, b: (b, s, 0)),
            pl.BlockSpec((BS, D_MODEL), lambda s, b: (s, 0)),
        ],
        out_specs=pl.BlockSpec((1, BS, D_MODEL), lambda s, b: (b, s, 0)),
        out_shape=jax.ShapeDtypeStruct((BATCH, SEQ_LEN, D_MODEL), jnp.float32),
    )(x, pos_table)
